# baseline (speedup 1.0000x reference)
import functools

import jax
import jax.numpy as jnp
from jax import lax
from jax.experimental import pallas as pl
from jax.experimental.pallas import tpu as pltpu
from jax.experimental.pallas import tpu_sc as plsc

_NC = 2
_NS = 16
_NW = _NC * _NS
_LANES = 16


def _tc_add_kernel(mid_ref, table_ref, x_ref, o_ref):
    row = table_ref[mid_ref[0], :]
    o_ref[...] = x_ref[...] + row[None, :]


def _tc_add(x2, modality_table, mid, block):
    rows, E = x2.shape
    grid = rows // block
    return pl.pallas_call(
        _tc_add_kernel,
        grid_spec=pltpu.PrefetchScalarGridSpec(
            num_scalar_prefetch=1,
            grid=(grid,),
            in_specs=[
                pl.BlockSpec(modality_table.shape, lambda i, m: (0, 0)),
                pl.BlockSpec((block, E), lambda i, m: (i, 0)),
            ],
            out_specs=pl.BlockSpec((block, E), lambda i, m: (i, 0)),
        ),
        out_shape=jax.ShapeDtypeStruct((rows, E), x2.dtype),
    )(mid, modality_table, x2)


def _sc_add(xf, modality_table, mid, chunk_rows):
    E = modality_table.shape[1]
    n = xf.shape[0] // E
    rows_per_w = n // _NW
    che = chunk_rows * E
    n_chunks = rows_per_w // chunk_rows
    depth = 4
    assert n_chunks % depth == 0

    mesh = plsc.VectorSubcoreMesh(core_axis_name="c", subcore_axis_name="s")

    scratch = (
        [pltpu.VMEM((1,), jnp.int32), pltpu.VMEM((1, E), jnp.float32)]
        + [pltpu.VMEM((che,), jnp.float32)]
        + [pltpu.VMEM((che,), jnp.float32)] * (2 * depth)
        + [pltpu.SemaphoreType.DMA] * (2 * depth)
    )

    @functools.partial(
        pl.kernel,
        mesh=mesh,
        out_type=jax.ShapeDtypeStruct((n * E,), jnp.float32),
        scratch_types=scratch,
    )
    def k(x_hbm, table_hbm, mid_hbm, out_hbm, idx_v, emb_v, rep_v, *rest):
        ib = rest[:depth]
        ob = rest[depth:2 * depth]
        si = rest[2 * depth:3 * depth]
        so = rest[3 * depth:4 * depth]
        wid = lax.axis_index("s") * _NC + lax.axis_index("c")
        pltpu.sync_copy(mid_hbm, idx_v)
        pltpu.async_copy(table_hbm.at[idx_v], emb_v, si[0]).wait()
        base = wid * rows_per_w * E

        @plsc.parallel_loop(0, che, _LANES, unroll=8)
        def _(p):
            e = (p & (E - 1))
            rep_v[pl.ds(p, _LANES)] = emb_v[0, pl.ds(e, _LANES)]

        def start_in(c, b):
            pltpu.make_async_copy(
                x_hbm.at[pl.ds(base + c * che, che)], ib[b], si[b]).start()

        def wait_in(c, b):
            pltpu.make_async_copy(
                x_hbm.at[pl.ds(base + c * che, che)], ib[b], si[b]).wait()

        def start_out(c, b):
            pltpu.make_async_copy(
                ob[b], out_hbm.at[pl.ds(base + c * che, che)], so[b]).start()

        def wait_out(c, b):
            pltpu.make_async_copy(
                ob[b], out_hbm.at[pl.ds(base + c * che, che)], so[b]).wait()

        for b in range(depth):
            start_in(b, b)

        @pl.loop(0, n_chunks, step=depth)
        def _(t):
            for b in range(depth):
                c = t + b
                wait_in(c, b)

                @pl.when(c >= depth)
                def _():
                    wait_out(c - depth, b)

                @plsc.parallel_loop(0, che, _LANES, unroll=8)
                def _(p):
                    ob[b][pl.ds(p, _LANES)] = (
                        ib[b][pl.ds(p, _LANES)] + rep_v[pl.ds(p, _LANES)]
                    )

                start_out(c, b)

                @pl.when(c + depth < n_chunks)
                def _():
                    start_in(c + depth, b)

        for b in range(depth):
            wait_out(n_chunks - depth + b, b)

    return k(xf, modality_table, mid)


def _sc_gather_row(modality_table, mid):
    E = modality_table.shape[1]
    mesh = plsc.ScalarSubcoreMesh(axis_name="c", num_cores=1)

    @functools.partial(
        pl.kernel,
        mesh=mesh,
        out_type=jax.ShapeDtypeStruct((1, E), jnp.float32),
        scratch_types=[
            pltpu.SMEM((1,), jnp.int32),
        ],
    )
    def k(table_hbm, mid_hbm, out_hbm, mid_s):
        pltpu.sync_copy(mid_hbm, mid_s)
        m = mid_s[0]
        pltpu.sync_copy(table_hbm.at[pl.ds(m, 1)], out_hbm)

    return k(modality_table, mid)


def _tc_row_add_kernel(row_ref, x_ref, o_ref):
    o_ref[...] = x_ref[...] + row_ref[0, :][None, :]


def _tc_row_add(x2, row, block):
    rows, E = x2.shape
    grid = rows // block
    return pl.pallas_call(
        _tc_row_add_kernel,
        grid=(grid,),
        in_specs=[
            pl.BlockSpec((1, E), lambda i: (0, 0)),
            pl.BlockSpec((block, E), lambda i: (i, 0)),
        ],
        out_specs=pl.BlockSpec((block, E), lambda i: (i, 0)),
        out_shape=jax.ShapeDtypeStruct((rows, E), x2.dtype),
    )(row, x2)


def kernel(x, modality_table, modality_id):
    B, S, E = x.shape
    rows = B * S
    mid = jnp.asarray(modality_id, jnp.int32).reshape((1,))
    row = _sc_gather_row(modality_table, mid)
    out = _tc_row_add(x.reshape(rows, E), row, 512)
    return out.reshape(B, S, E)


# speedup vs baseline: 1.0140x; 1.0140x over previous
import functools

import jax
import jax.numpy as jnp
from jax.experimental import pallas as pl
from jax.experimental.pallas import tpu as pltpu
from jax.experimental.pallas import tpu_sc as plsc


def _sc_gather_row(modality_table, mid):
    E = modality_table.shape[1]
    mesh = plsc.ScalarSubcoreMesh(axis_name="c", num_cores=1)

    @functools.partial(
        pl.kernel,
        mesh=mesh,
        out_type=jax.ShapeDtypeStruct((1, E), jnp.float32),
        scratch_types=[
            pltpu.SMEM((1,), jnp.int32),
        ],
    )
    def k(table_hbm, mid_hbm, out_hbm, mid_s):
        pltpu.sync_copy(mid_hbm, mid_s)
        m = mid_s[0]
        pltpu.sync_copy(table_hbm.at[pl.ds(m, 1)], out_hbm)

    return k(modality_table, mid)


def _tc_row_add_kernel(row_ref, x_ref, o_ref):
    o_ref[...] = x_ref[...] + row_ref[0, :][None, :]


def _tc_row_add(x2, row, block):
    rows, E = x2.shape
    grid = rows // block
    return pl.pallas_call(
        _tc_row_add_kernel,
        grid=(grid,),
        in_specs=[
            pl.BlockSpec((1, E), lambda i: (0, 0)),
            pl.BlockSpec((block, E), lambda i: (i, 0)),
        ],
        out_specs=pl.BlockSpec((block, E), lambda i: (i, 0)),
        out_shape=jax.ShapeDtypeStruct((rows, E), x2.dtype),
    )(row, x2)


def kernel(x, modality_table, modality_id):
    B, S, E = x.shape
    rows = B * S
    mid = jnp.asarray(modality_id, jnp.int32).reshape((1,))
    row = _sc_gather_row(modality_table, mid)
    out = _tc_row_add(x.reshape(rows, E), row, 1024)
    return out.reshape(B, S, E)


# speedup vs baseline: 1.0175x; 1.0035x over previous
import functools

import jax
import jax.numpy as jnp
from jax.experimental import pallas as pl
from jax.experimental.pallas import tpu as pltpu
from jax.experimental.pallas import tpu_sc as plsc


def _sc_gather_row(modality_table, mid):
    E = modality_table.shape[1]
    mesh = plsc.ScalarSubcoreMesh(axis_name="c", num_cores=1)

    @functools.partial(
        pl.kernel,
        mesh=mesh,
        out_type=jax.ShapeDtypeStruct((1, E), jnp.float32),
        scratch_types=[
            pltpu.SMEM((1,), jnp.int32),
        ],
    )
    def k(table_hbm, mid_hbm, out_hbm, mid_s):
        pltpu.sync_copy(mid_hbm, mid_s)
        m = mid_s[0]
        pltpu.sync_copy(table_hbm.at[pl.ds(m, 1)], out_hbm)

    return k(modality_table, mid)


def _tc_row_add_kernel(row_ref, x_ref, o_ref):
    o_ref[...] = x_ref[...] + row_ref[0, :][None, :]


def _tc_row_add(x2, row, block):
    rows, E = x2.shape
    grid = pl.cdiv(rows, block)
    return pl.pallas_call(
        _tc_row_add_kernel,
        grid=(grid,),
        in_specs=[
            pl.BlockSpec((1, E), lambda i: (0, 0)),
            pl.BlockSpec((block, E), lambda i: (i, 0)),
        ],
        out_specs=pl.BlockSpec((block, E), lambda i: (i, 0)),
        out_shape=jax.ShapeDtypeStruct((rows, E), x2.dtype),
    )(row, x2)


def kernel(x, modality_table, modality_id):
    B, S, E = x.shape
    rows = B * S
    mid = jnp.asarray(modality_id, jnp.int32).reshape((1,))
    row = _sc_gather_row(modality_table, mid)
    out = _tc_row_add(x.reshape(rows, E), row, 1792)
    return out.reshape(B, S, E)
